# baseline (speedup 1.0000x reference)
import functools

import jax
import jax.numpy as jnp
from jax import lax
from jax.experimental import pallas as pl
from jax.experimental.pallas import tpu as pltpu
from jax.experimental.pallas import tpu_sc as plsc

NUM_ROWS = 16384
NUM_CLASSES = 1000
LANES = 16
FULL_CHUNKS = NUM_CLASSES // LANES
TAIL_OFF = NUM_CLASSES - LANES
TAIL_START = FULL_CHUNKS * LANES
STRIDE = LANES + 1

_info = plsc.get_sparse_core_info()
NC, NS = _info.num_cores, _info.num_subcores
NW = NC * NS
RPW = NUM_ROWS // NW
RC = 16
NCHUNK = RPW // RC


def _make_kernel():
    mesh = plsc.VectorSubcoreMesh(core_axis_name="c", subcore_axis_name="s")

    @functools.partial(
        pl.kernel,
        mesh=mesh,
        out_type=jax.ShapeDtypeStruct((NUM_ROWS,), jnp.float32),
        compiler_params=pltpu.CompilerParams(needs_layout_passes=False),
        scratch_types=[
            pltpu.VMEM((RC, NUM_CLASSES), jnp.float32),
            pltpu.VMEM((RC, NUM_CLASSES), jnp.float32),
            pltpu.VMEM((RC * STRIDE,), jnp.float32),
            pltpu.VMEM((RPW,), jnp.float32),
            pltpu.SemaphoreType.DMA,
            pltpu.SemaphoreType.DMA,
        ],
    )
    def body(x_hbm, out_hbm, buf_a, buf_b, staging, out_v, sem_a, sem_b):
        wid = lax.axis_index("s") * NC + lax.axis_index("c")
        base = wid * RPW

        iota_i = lax.iota(jnp.int32, LANES)
        iota_f = iota_i.astype(jnp.float32)
        tail_mask = iota_i >= (TAIL_START - TAIL_OFF)
        zeros = jnp.zeros((LANES,), jnp.float32)

        def start(chunk, buf, sem):
            src = x_hbm.at[pl.ds(base + chunk * RC, RC)]
            pltpu.make_async_copy(src, buf, sem).start()

        def wait(chunk, buf, sem):
            src = x_hbm.at[pl.ds(base + chunk * RC, RC)]
            pltpu.make_async_copy(src, buf, sem).wait()

        def compute(buf, chunk):
            def row_body(r, _):
                a = buf[r, pl.ds(0, LANES)]
                b = zeros
                for c in range(1, FULL_CHUNKS):
                    x = buf[r, pl.ds(c * LANES, LANES)]
                    a = a + x
                    b = b + x * float(c * LANES)
                tail = buf[r, pl.ds(TAIL_OFF, LANES)]
                xm = jnp.where(tail_mask, tail, 0.0)
                a = a + xm
                b = b + xm * float(TAIL_OFF)
                staging[pl.ds(r * STRIDE, LANES)] = b + a * iota_f
                return 0

            lax.fori_loop(0, RC, row_body, 0)

            sums = plsc.load_gather(staging, [iota_i * STRIDE])
            for c in range(1, LANES):
                sums = sums + plsc.load_gather(staging, [iota_i * STRIDE + c])
            out_v[pl.ds(chunk * RC, RC)] = sums

        start(0, buf_a, sem_a)

        def loop_body(g, _):
            c0 = 2 * g
            c1 = 2 * g + 1
            c2 = jnp.minimum(2 * g + 2, NCHUNK - 1)
            start(c1, buf_b, sem_b)
            wait(c0, buf_a, sem_a)
            compute(buf_a, c0)
            start(c2, buf_a, sem_a)
            wait(c1, buf_b, sem_b)
            compute(buf_b, c1)
            return 0

        lax.fori_loop(0, NCHUNK // 2, loop_body, 0)
        wait(NCHUNK - 1, buf_a, sem_a)

        pltpu.sync_copy(out_v, out_hbm.at[pl.ds(base, RPW)])

    return body


_sc_kernel = _make_kernel()


def kernel(onehot):
    return _sc_kernel(onehot).astype(jnp.int32)


# speedup vs baseline: 1.0726x; 1.0726x over previous
import functools

import jax
import jax.numpy as jnp
from jax import lax
from jax.experimental import pallas as pl
from jax.experimental.pallas import tpu as pltpu
from jax.experimental.pallas import tpu_sc as plsc

NUM_ROWS = 16384
NUM_CLASSES = 1000
LANES = 16
FULL_CHUNKS = NUM_CLASSES // LANES
TAIL_OFF = NUM_CLASSES - LANES
TAIL_START = FULL_CHUNKS * LANES
STRIDE = LANES + 1

_info = plsc.get_sparse_core_info()
NC, NS = _info.num_cores, _info.num_subcores
NW = NC * NS
RPW = NUM_ROWS // NW
RC = 16
NCHUNK = RPW // RC


def _make_kernel():
    mesh = plsc.VectorSubcoreMesh(core_axis_name="c", subcore_axis_name="s")

    @functools.partial(
        pl.kernel,
        mesh=mesh,
        out_type=jax.ShapeDtypeStruct((NUM_ROWS,), jnp.float32),
        compiler_params=pltpu.CompilerParams(needs_layout_passes=False),
        scratch_types=[
            pltpu.VMEM((RC, NUM_CLASSES), jnp.float32),
            pltpu.VMEM((RC, NUM_CLASSES), jnp.float32),
            pltpu.VMEM((RC * STRIDE,), jnp.float32),
            pltpu.VMEM((RPW,), jnp.float32),
            pltpu.SemaphoreType.DMA,
            pltpu.SemaphoreType.DMA,
        ],
    )
    def body(x_hbm, out_hbm, buf_a, buf_b, staging, out_v, sem_a, sem_b):
        wid = lax.axis_index("s") * NC + lax.axis_index("c")
        base = wid * RPW

        iota_i = lax.iota(jnp.int32, LANES)
        iota_f = iota_i.astype(jnp.float32)
        tail_mask = iota_i >= (TAIL_START - TAIL_OFF)
        zeros = jnp.zeros((LANES,), jnp.float32)

        def start(chunk, buf, sem):
            src = x_hbm.at[pl.ds(base + chunk * RC, RC)]
            pltpu.make_async_copy(src, buf, sem).start()

        def wait(chunk, buf, sem):
            src = x_hbm.at[pl.ds(base + chunk * RC, RC)]
            pltpu.make_async_copy(src, buf, sem).wait()

        def _tree_sum(vs):
            while len(vs) > 1:
                vs = [x + y for x, y in zip(vs[::2], vs[1::2])] + (
                    [vs[-1]] if len(vs) % 2 else [])
            return vs[0]

        NACC = 8

        def compute(buf, chunk):
            def row_body(r, _):
                a = [None] * NACC
                b = [None] * NACC
                for c in range(FULL_CHUNKS):
                    x = buf[r, pl.ds(c * LANES, LANES)]
                    i = c % NACC
                    a[i] = x if a[i] is None else a[i] + x
                    if c:
                        t = x * float(c * LANES)
                        b[i] = t if b[i] is None else b[i] + t
                tail = buf[r, pl.ds(TAIL_OFF, LANES)]
                xm = jnp.where(tail_mask, tail, 0.0)
                a[NACC - 1] = a[NACC - 1] + xm
                b[NACC - 1] = b[NACC - 1] + xm * float(TAIL_OFF)
                av = _tree_sum([v for v in a if v is not None])
                bv = _tree_sum([v for v in b if v is not None])
                staging[pl.ds(r * STRIDE, LANES)] = bv + av * iota_f
                return 0

            lax.fori_loop(0, RC, row_body, 0)

            sums = plsc.load_gather(staging, [iota_i * STRIDE])
            for c in range(1, LANES):
                sums = sums + plsc.load_gather(staging, [iota_i * STRIDE + c])
            out_v[pl.ds(chunk * RC, RC)] = sums

        start(0, buf_a, sem_a)

        def loop_body(g, _):
            c0 = 2 * g
            c1 = 2 * g + 1
            c2 = jnp.minimum(2 * g + 2, NCHUNK - 1)
            start(c1, buf_b, sem_b)
            wait(c0, buf_a, sem_a)
            compute(buf_a, c0)
            start(c2, buf_a, sem_a)
            wait(c1, buf_b, sem_b)
            compute(buf_b, c1)
            return 0

        lax.fori_loop(0, NCHUNK // 2, loop_body, 0)
        wait(NCHUNK - 1, buf_a, sem_a)

        pltpu.sync_copy(out_v, out_hbm.at[pl.ds(base, RPW)])

    return body


_sc_kernel = _make_kernel()


def kernel(onehot):
    return _sc_kernel(onehot).astype(jnp.int32)


# speedup vs baseline: 2.3125x; 2.1561x over previous
import functools

import jax
import jax.numpy as jnp
from jax import lax
from jax.experimental import pallas as pl
from jax.experimental.pallas import tpu as pltpu
from jax.experimental.pallas import tpu_sc as plsc

NUM_ROWS = 16384
NUM_CLASSES = 1000
LANES = 16
TILE_W = 128
GROUPS = TILE_W // LANES

_info = plsc.get_sparse_core_info()
NC, NS = _info.num_cores, _info.num_subcores
NW = NC * NS
SPW = NUM_ROWS // NW
TCPW = SPW // TILE_W
CT_PER_CHUNK = 25
CLS_PER_CHUNK = CT_PER_CHUNK * 8
PARTS = NUM_CLASSES // CLS_PER_CHUNK
NCHUNK = TCPW * PARTS


def _make_kernel():
    mesh = plsc.VectorSubcoreMesh(core_axis_name="c", subcore_axis_name="s")

    @functools.partial(
        pl.kernel,
        mesh=mesh,
        out_type=jax.ShapeDtypeStruct((NUM_ROWS,), jnp.float32),
        compiler_params=pltpu.CompilerParams(
            needs_layout_passes=False, use_tc_tiling_on_sc=True),
        scratch_types=[
            pltpu.VMEM((CLS_PER_CHUNK, TILE_W), jnp.float32),
            pltpu.VMEM((CLS_PER_CHUNK, TILE_W), jnp.float32),
            pltpu.VMEM((SPW,), jnp.float32),
            pltpu.SemaphoreType.DMA,
            pltpu.SemaphoreType.DMA,
        ],
    )
    def body(xt_hbm, out_hbm, buf_a, buf_b, out_v, sem_a, sem_b):
        wid = lax.axis_index("s") * NC + lax.axis_index("c")
        sample_base = wid * SPW

        zeros = jnp.zeros((LANES,), jnp.float32)
        for i in range(SPW // LANES):
            out_v[pl.ds(i * LANES, LANES)] = zeros

        def chunk_src(c, buf, sem):
            tc_l = c // PARTS
            p = c % PARTS
            src = xt_hbm.at[
                pl.ds(p * CLS_PER_CHUNK, CLS_PER_CHUNK),
                pl.ds(sample_base + tc_l * TILE_W, TILE_W),
            ]
            return pltpu.make_async_copy(src, buf, sem)

        def compute(buf, c):
            tc_l = c // PARTS
            cls0 = (c % PARTS) * CLS_PER_CHUNK
            col0 = tc_l * TILE_W

            def t_body(t, accs):
                base = (cls0 + t * 8).astype(jnp.float32)
                out = list(accs)
                for s in range(8):
                    w = base + float(s)
                    for g in range(GROUPS):
                        x = buf[t * 8 + s, pl.ds(g * LANES, LANES)]
                        out[g] = out[g] + x * w
                return tuple(out)

            accs = tuple(
                out_v[pl.ds(col0 + g * LANES, LANES)] for g in range(GROUPS))
            accs = lax.fori_loop(0, CT_PER_CHUNK, t_body, accs)
            for g in range(GROUPS):
                out_v[pl.ds(col0 + g * LANES, LANES)] = accs[g]

        chunk_src(0, buf_a, sem_a).start()

        def loop_body(i, _):
            c0 = 2 * i
            c1 = 2 * i + 1
            c2 = jnp.minimum(2 * i + 2, NCHUNK - 1)
            chunk_src(c1, buf_b, sem_b).start()
            chunk_src(c0, buf_a, sem_a).wait()
            compute(buf_a, c0)
            chunk_src(c2, buf_a, sem_a).start()
            chunk_src(c1, buf_b, sem_b).wait()
            compute(buf_b, c1)
            return 0

        lax.fori_loop(0, NCHUNK // 2, loop_body, 0)
        chunk_src(NCHUNK - 1, buf_a, sem_a).wait()

        pltpu.sync_copy(out_v, out_hbm.at[pl.ds(sample_base, SPW)])

    return body


_sc_kernel = _make_kernel()


def kernel(onehot):
    return _sc_kernel(onehot.T).astype(jnp.int32)


# speedup vs baseline: 2.6223x; 1.1340x over previous
import functools

import jax
import jax.numpy as jnp
from jax import lax
from jax.experimental import pallas as pl
from jax.experimental.pallas import tpu as pltpu
from jax.experimental.pallas import tpu_sc as plsc

NUM_ROWS = 16384
NUM_CLASSES = 1000
LANES = 16
TILE_W = 128
GROUPS = TILE_W // LANES

_info = plsc.get_sparse_core_info()
NC, NS = _info.num_cores, _info.num_subcores
NW = NC * NS

SC_SAMPLES = 4096
TC_SAMPLES = NUM_ROWS - SC_SAMPLES
SC_BASE = TC_SAMPLES
SPW = SC_SAMPLES // NW
CT_PER_CHUNK = 25
CLS_PER_CHUNK = CT_PER_CHUNK * 8
PARTS = NUM_CLASSES // CLS_PER_CHUNK
assert PARTS % 2 == 1


def _make_sc_kernel():
    mesh = plsc.VectorSubcoreMesh(core_axis_name="c", subcore_axis_name="s")

    @functools.partial(
        pl.kernel,
        mesh=mesh,
        out_type=jax.ShapeDtypeStruct((SC_SAMPLES,), jnp.float32),
        compiler_params=pltpu.CompilerParams(
            needs_layout_passes=False, use_tc_tiling_on_sc=True),
        scratch_types=[
            pltpu.VMEM((CLS_PER_CHUNK, TILE_W), jnp.float32),
            pltpu.VMEM((CLS_PER_CHUNK, TILE_W), jnp.float32),
            pltpu.VMEM((SPW,), jnp.float32),
            pltpu.SemaphoreType.DMA,
            pltpu.SemaphoreType.DMA,
        ],
    )
    def body(xt_hbm, out_hbm, buf_a, buf_b, out_v, sem_a, sem_b):
        wid = lax.axis_index("s") * NC + lax.axis_index("c")
        col_base = SC_BASE + wid * SPW

        zeros = jnp.zeros((LANES,), jnp.float32)
        for i in range(SPW // LANES):
            out_v[pl.ds(i * LANES, LANES)] = zeros

        def chunk_src(p, buf, sem):
            src = xt_hbm.at[
                pl.ds(p * CLS_PER_CHUNK, CLS_PER_CHUNK),
                pl.ds(col_base, TILE_W),
            ]
            return pltpu.make_async_copy(src, buf, sem)

        def compute(buf, p):
            cls0 = p * CLS_PER_CHUNK

            def t_body(t, accs):
                base = (cls0 + t * 8).astype(jnp.float32)
                out = list(accs)
                for s in range(8):
                    w = base + float(s)
                    for g in range(GROUPS):
                        x = buf[t * 8 + s, pl.ds(g * LANES, LANES)]
                        out[g] = out[g] + x * w
                return tuple(out)

            accs = tuple(
                out_v[pl.ds(g * LANES, LANES)] for g in range(GROUPS))
            accs = lax.fori_loop(0, CT_PER_CHUNK, t_body, accs)
            for g in range(GROUPS):
                out_v[pl.ds(g * LANES, LANES)] = accs[g]

        chunk_src(0, buf_a, sem_a).start()

        def loop_body(i, _):
            c0 = 2 * i
            c1 = 2 * i + 1
            chunk_src(c1, buf_b, sem_b).start()
            chunk_src(c0, buf_a, sem_a).wait()
            compute(buf_a, c0)
            chunk_src(jnp.minimum(c1 + 1, PARTS - 1), buf_a, sem_a).start()
            chunk_src(c1, buf_b, sem_b).wait()
            compute(buf_b, c1)
            return 0

        lax.fori_loop(0, PARTS // 2, loop_body, 0)
        chunk_src(PARTS - 1, buf_a, sem_a).wait()
        compute(buf_a, PARTS - 1)

        pltpu.sync_copy(out_v, out_hbm.at[pl.ds(wid * SPW, SPW)])

    return body


TC_SB = 2048
TC_CB = 200
TC_GRID = (TC_SAMPLES // TC_SB, NUM_CLASSES // TC_CB)


def _tc_body(x_ref, o_ref):
    cb = pl.program_id(1)
    w = lax.broadcasted_iota(jnp.int32, (TC_CB, 1), 0).astype(jnp.float32) + (
        cb * TC_CB).astype(jnp.float32)
    part = jnp.sum(x_ref[...] * w, axis=0)

    @pl.when(cb == 0)
    def _init():
        o_ref[...] = part

    @pl.when(cb != 0)
    def _acc():
        o_ref[...] = o_ref[...] + part


_tc_kernel = pl.pallas_call(
    _tc_body,
    grid=TC_GRID,
    in_specs=[pl.BlockSpec((TC_CB, TC_SB), lambda i, j: (j, i))],
    out_specs=pl.BlockSpec((TC_SB,), lambda i, j: (i,)),
    out_shape=jax.ShapeDtypeStruct((TC_SAMPLES,), jnp.float32),
    compiler_params=pltpu.CompilerParams(
        dimension_semantics=("parallel", "arbitrary")),
)

_sc_kernel = _make_sc_kernel()


def kernel(onehot):
    xt = onehot.T
    sc_out = _sc_kernel(xt)
    tc_out = _tc_kernel(xt)
    return jnp.concatenate([tc_out, sc_out]).astype(jnp.int32)


# speedup vs baseline: 3.1097x; 1.1859x over previous
import functools

import jax
import jax.numpy as jnp
from jax import lax
from jax.experimental import pallas as pl
from jax.experimental.pallas import tpu as pltpu
from jax.experimental.pallas import tpu_sc as plsc

NUM_ROWS = 16384
NUM_CLASSES = 1000
LANES = 16
TILE_W = 128
GROUPS = TILE_W // LANES

_info = plsc.get_sparse_core_info()
NC, NS = _info.num_cores, _info.num_subcores
NW = NC * NS

SC_SAMPLES = 4096
TC_SAMPLES = NUM_ROWS - SC_SAMPLES
SC_BASE = TC_SAMPLES
SPW = SC_SAMPLES // NW
CT_PER_CHUNK = 25
CLS_PER_CHUNK = CT_PER_CHUNK * 8
PARTS = NUM_CLASSES // CLS_PER_CHUNK
assert PARTS % 2 == 1


def _make_sc_kernel():
    mesh = plsc.VectorSubcoreMesh(core_axis_name="c", subcore_axis_name="s")

    @functools.partial(
        pl.kernel,
        mesh=mesh,
        out_type=jax.ShapeDtypeStruct((SC_SAMPLES,), jnp.float32),
        compiler_params=pltpu.CompilerParams(
            needs_layout_passes=False, use_tc_tiling_on_sc=True),
        scratch_types=[
            pltpu.VMEM((CLS_PER_CHUNK, TILE_W), jnp.float32),
            pltpu.VMEM((CLS_PER_CHUNK, TILE_W), jnp.float32),
            pltpu.VMEM((SPW,), jnp.float32),
            pltpu.SemaphoreType.DMA,
            pltpu.SemaphoreType.DMA,
        ],
    )
    def body(xt_hbm, out_hbm, buf_a, buf_b, out_v, sem_a, sem_b):
        wid = lax.axis_index("s") * NC + lax.axis_index("c")
        col_base = SC_BASE + wid * SPW

        zeros = jnp.zeros((LANES,), jnp.float32)
        for i in range(SPW // LANES):
            out_v[pl.ds(i * LANES, LANES)] = zeros

        def chunk_src(p, buf, sem):
            src = xt_hbm.at[
                pl.ds(p * CLS_PER_CHUNK, CLS_PER_CHUNK),
                pl.ds(col_base, TILE_W),
            ]
            return pltpu.make_async_copy(src, buf, sem)

        def compute(buf, p):
            cls0 = p * CLS_PER_CHUNK

            def t_body(t, accs):
                out = list(accs)
                for s in range(2):
                    w = (cls0 + 2 * t + s).astype(jnp.float32)
                    for g in range(GROUPS):
                        x = buf[2 * t + s, pl.ds(g * LANES, LANES)]
                        out[g] = out[g] + x * w
                return tuple(out)

            accs = tuple(
                out_v[pl.ds(g * LANES, LANES)] for g in range(GROUPS))
            accs = lax.fori_loop(0, CLS_PER_CHUNK // 2, t_body, accs)
            for g in range(GROUPS):
                out_v[pl.ds(g * LANES, LANES)] = accs[g]

        chunk_src(0, buf_a, sem_a).start()

        def loop_body(i, _):
            c0 = 2 * i
            c1 = 2 * i + 1
            chunk_src(c1, buf_b, sem_b).start()
            chunk_src(c0, buf_a, sem_a).wait()
            compute(buf_a, c0)
            chunk_src(jnp.minimum(c1 + 1, PARTS - 1), buf_a, sem_a).start()
            chunk_src(c1, buf_b, sem_b).wait()
            compute(buf_b, c1)
            return 0

        lax.fori_loop(0, PARTS // 2, loop_body, 0)
        chunk_src(PARTS - 1, buf_a, sem_a).wait()
        compute(buf_a, PARTS - 1)

        pltpu.sync_copy(out_v, out_hbm.at[pl.ds(wid * SPW, SPW)])

    return body


TC_SB = 2048
TC_GRID = (TC_SAMPLES // TC_SB,)


def _tc_body(x_ref, o_ref):
    w = lax.broadcasted_iota(
        jnp.int32, (NUM_CLASSES, 1), 0).astype(jnp.float32)
    o_ref[...] = jnp.sum(x_ref[...] * w, axis=0)


_tc_kernel = pl.pallas_call(
    _tc_body,
    grid=TC_GRID,
    in_specs=[pl.BlockSpec((NUM_CLASSES, TC_SB), lambda i: (0, i))],
    out_specs=pl.BlockSpec((TC_SB,), lambda i: (i,)),
    out_shape=jax.ShapeDtypeStruct((TC_SAMPLES,), jnp.float32),
    compiler_params=pltpu.CompilerParams(
        dimension_semantics=("parallel",)),
)

_sc_kernel = _make_sc_kernel()


def kernel(onehot):
    xt = onehot.T
    sc_out = _sc_kernel(xt)
    tc_out = _tc_kernel(xt)
    return jnp.concatenate([tc_out, sc_out]).astype(jnp.int32)


# speedup vs baseline: 3.1155x; 1.0019x over previous
import functools

import jax
import jax.numpy as jnp
from jax import lax
from jax.experimental import pallas as pl
from jax.experimental.pallas import tpu as pltpu
from jax.experimental.pallas import tpu_sc as plsc

NUM_ROWS = 16384
NUM_CLASSES = 1000
LANES = 16
TILE_W = 128
GROUPS = TILE_W // LANES

_info = plsc.get_sparse_core_info()
NC, NS = _info.num_cores, _info.num_subcores
NW = NC * NS

SC_SAMPLES = 4096
TC_SAMPLES = NUM_ROWS - SC_SAMPLES
SC_BASE = TC_SAMPLES
SPW = SC_SAMPLES // NW
CT_PER_CHUNK = 25
CLS_PER_CHUNK = CT_PER_CHUNK * 8
PARTS = NUM_CLASSES // CLS_PER_CHUNK
assert PARTS % 2 == 1


def _make_sc_kernel():
    mesh = plsc.VectorSubcoreMesh(core_axis_name="c", subcore_axis_name="s")

    @functools.partial(
        pl.kernel,
        mesh=mesh,
        out_type=jax.ShapeDtypeStruct((SC_SAMPLES,), jnp.float32),
        compiler_params=pltpu.CompilerParams(
            needs_layout_passes=False, use_tc_tiling_on_sc=True,
            skip_device_barrier=True),
        scratch_types=[
            pltpu.VMEM((CLS_PER_CHUNK, TILE_W), jnp.float32),
            pltpu.VMEM((CLS_PER_CHUNK, TILE_W), jnp.float32),
            pltpu.VMEM((SPW,), jnp.float32),
            pltpu.SemaphoreType.DMA,
            pltpu.SemaphoreType.DMA,
        ],
    )
    def body(xt_hbm, out_hbm, buf_a, buf_b, out_v, sem_a, sem_b):
        wid = lax.axis_index("s") * NC + lax.axis_index("c")
        col_base = SC_BASE + wid * SPW

        zeros = jnp.zeros((LANES,), jnp.float32)
        for i in range(SPW // LANES):
            out_v[pl.ds(i * LANES, LANES)] = zeros

        def chunk_src(p, buf, sem):
            src = xt_hbm.at[
                pl.ds(p * CLS_PER_CHUNK, CLS_PER_CHUNK),
                pl.ds(col_base, TILE_W),
            ]
            return pltpu.make_async_copy(src, buf, sem)

        def compute(buf, p):
            cls0 = p * CLS_PER_CHUNK

            def t_body(t, accs):
                out = list(accs)
                for s in range(2):
                    w = (cls0 + 2 * t + s).astype(jnp.float32)
                    for g in range(GROUPS):
                        x = buf[2 * t + s, pl.ds(g * LANES, LANES)]
                        out[g] = out[g] + x * w
                return tuple(out)

            accs = tuple(
                out_v[pl.ds(g * LANES, LANES)] for g in range(GROUPS))
            accs = lax.fori_loop(0, CLS_PER_CHUNK // 2, t_body, accs)
            for g in range(GROUPS):
                out_v[pl.ds(g * LANES, LANES)] = accs[g]

        chunk_src(0, buf_a, sem_a).start()

        def loop_body(i, _):
            c0 = 2 * i
            c1 = 2 * i + 1
            chunk_src(c1, buf_b, sem_b).start()
            chunk_src(c0, buf_a, sem_a).wait()
            compute(buf_a, c0)
            chunk_src(jnp.minimum(c1 + 1, PARTS - 1), buf_a, sem_a).start()
            chunk_src(c1, buf_b, sem_b).wait()
            compute(buf_b, c1)
            return 0

        lax.fori_loop(0, PARTS // 2, loop_body, 0)
        chunk_src(PARTS - 1, buf_a, sem_a).wait()
        compute(buf_a, PARTS - 1)

        pltpu.sync_copy(out_v, out_hbm.at[pl.ds(wid * SPW, SPW)])

    return body


TC_SB = 2048
TC_GRID = (TC_SAMPLES // TC_SB,)


def _tc_body(x_ref, o_ref):
    w = lax.broadcasted_iota(
        jnp.int32, (NUM_CLASSES, 1), 0).astype(jnp.float32)
    o_ref[...] = jnp.sum(x_ref[...] * w, axis=0)


_tc_kernel = pl.pallas_call(
    _tc_body,
    grid=TC_GRID,
    in_specs=[pl.BlockSpec((NUM_CLASSES, TC_SB), lambda i: (0, i))],
    out_specs=pl.BlockSpec((TC_SB,), lambda i: (i,)),
    out_shape=jax.ShapeDtypeStruct((TC_SAMPLES,), jnp.float32),
    compiler_params=pltpu.CompilerParams(
        dimension_semantics=("parallel",), skip_device_barrier=True),
)

_sc_kernel = _make_sc_kernel()


def kernel(onehot):
    xt = onehot.T
    sc_out = _sc_kernel(xt)
    tc_out = _tc_kernel(xt)
    return jnp.concatenate([tc_out, sc_out]).astype(jnp.int32)
